# baseline (speedup 1.0000x reference)
import jax
import jax.numpy as jnp
from jax import lax
from jax.experimental import pallas as pl
from jax.experimental.pallas import tpu as pltpu

_BM = 512


def _router_body(h_ref, w_ref, bt_ref, logits_ref, probs_ref, depth_ref):
    h = h_ref[...]
    w = w_ref[...]
    bt = bt_ref[...]
    logits = lax.dot_general(w, h, (((0,), (1,)), ((), ())),
                             preferred_element_type=jnp.float32) + bt
    logits_ref[...] = logits
    m = jnp.max(logits, axis=0, keepdims=True)
    e = jnp.exp(logits - m)
    s = jnp.sum(e, axis=0, keepdims=True)
    probs_ref[...] = e / s
    c = logits.shape[0]
    iota = lax.broadcasted_iota(jnp.int32, logits.shape, 0)
    cand = jnp.where(logits == m, iota, c)
    idx = jnp.min(cand, axis=0, keepdims=True)
    depth_ref[...] = idx + 1


def kernel(hidden_states, W, b):
    B, S, D = hidden_states.shape
    C = W.shape[-1]
    N = B * S
    h2 = hidden_states.reshape(N, D)
    bt = b.reshape(C, 1)

    grid = (N // _BM,)
    logitsT, probsT, depthT = pl.pallas_call(
        _router_body,
        grid=grid,
        in_specs=[
            pl.BlockSpec((_BM, D), lambda i: (i, 0)),
            pl.BlockSpec((D, C), lambda i: (0, 0)),
            pl.BlockSpec((C, 1), lambda i: (0, 0)),
        ],
        out_specs=[
            pl.BlockSpec((C, _BM), lambda i: (0, i)),
            pl.BlockSpec((C, _BM), lambda i: (0, i)),
            pl.BlockSpec((1, _BM), lambda i: (0, i)),
        ],
        out_shape=[
            jax.ShapeDtypeStruct((C, N), jnp.float32),
            jax.ShapeDtypeStruct((C, N), jnp.float32),
            jax.ShapeDtypeStruct((1, N), jnp.int32),
        ],
        compiler_params=pltpu.CompilerParams(
            dimension_semantics=("arbitrary",),
        ),
    )(h2, W, bt)

    depth_values = depthT.reshape(B, S)
    last_loss = jnp.zeros((), dtype=jnp.float32)
    return (depth_values, probsT.T.reshape(B, S, C),
            logitsT.T.reshape(B, S, C), last_loss)


# speedup vs baseline: 1.1713x; 1.1713x over previous
import jax
import jax.numpy as jnp
from jax import lax
from jax.experimental import pallas as pl
from jax.experimental.pallas import tpu as pltpu

_BM = 2048


def _router_body(h_ref, w_ref, bt_ref, logits_ref, probs_ref, depth_ref):
    h = h_ref[...]
    w = w_ref[...]
    bt = bt_ref[...]
    logits = lax.dot_general(w, h, (((0,), (1,)), ((), ())),
                             preferred_element_type=jnp.float32) + bt
    logits_ref[...] = logits
    m = jnp.max(logits, axis=0, keepdims=True)
    e = jnp.exp(logits - m)
    s = jnp.sum(e, axis=0, keepdims=True)
    probs_ref[...] = e / s
    c = logits.shape[0]
    iota = lax.broadcasted_iota(jnp.int32, logits.shape, 0)
    cand = jnp.where(logits == m, iota, c)
    idx = jnp.min(cand, axis=0, keepdims=True)
    depth_ref[...] = idx + 1


def kernel(hidden_states, W, b):
    B, S, D = hidden_states.shape
    C = W.shape[-1]
    N = B * S
    h2 = hidden_states.reshape(N, D)
    bt = b.reshape(C, 1)

    grid = (N // _BM,)
    logitsT, probsT, depthT = pl.pallas_call(
        _router_body,
        grid=grid,
        in_specs=[
            pl.BlockSpec((_BM, D), lambda i: (i, 0)),
            pl.BlockSpec((D, C), lambda i: (0, 0)),
            pl.BlockSpec((C, 1), lambda i: (0, 0)),
        ],
        out_specs=[
            pl.BlockSpec((C, _BM), lambda i: (0, i)),
            pl.BlockSpec((C, _BM), lambda i: (0, i)),
            pl.BlockSpec((1, _BM), lambda i: (0, i)),
        ],
        out_shape=[
            jax.ShapeDtypeStruct((C, N), jnp.float32),
            jax.ShapeDtypeStruct((C, N), jnp.float32),
            jax.ShapeDtypeStruct((1, N), jnp.int32),
        ],
        compiler_params=pltpu.CompilerParams(
            dimension_semantics=("arbitrary",),
        ),
    )(h2, W, bt)

    depth_values = depthT.reshape(B, S)
    last_loss = jnp.zeros((), dtype=jnp.float32)
    return (depth_values, probsT.T.reshape(B, S, C),
            logitsT.T.reshape(B, S, C), last_loss)


# speedup vs baseline: 1.1896x; 1.0156x over previous
import jax
import jax.numpy as jnp
from jax import lax
from jax.experimental import pallas as pl
from jax.experimental.pallas import tpu as pltpu

_BM = 1024


def _router_body(h_ref, w_ref, bt_ref, logits_ref, probs_ref, depth_ref):
    h = h_ref[...]
    w = w_ref[...]
    bt = bt_ref[...]
    logits = lax.dot_general(w, h, (((0,), (1,)), ((), ())),
                             preferred_element_type=jnp.float32) + bt
    logits_ref[...] = logits
    m = jnp.max(logits, axis=0, keepdims=True)
    e = jnp.exp(logits - m)
    s = jnp.sum(e, axis=0, keepdims=True)
    probs_ref[...] = e / s
    c = logits.shape[0]
    iota = lax.broadcasted_iota(jnp.int32, logits.shape, 0)
    cand = jnp.where(logits == m, iota, c)
    idx = jnp.min(cand, axis=0, keepdims=True)
    depth_ref[...] = idx + 1


def kernel(hidden_states, W, b):
    B, S, D = hidden_states.shape
    C = W.shape[-1]
    N = B * S
    h2 = hidden_states.reshape(N, D)
    bt = b.reshape(C, 1)

    grid = (N // _BM,)
    logitsT, probsT, depthT = pl.pallas_call(
        _router_body,
        grid=grid,
        in_specs=[
            pl.BlockSpec((_BM, D), lambda i: (i, 0)),
            pl.BlockSpec((D, C), lambda i: (0, 0)),
            pl.BlockSpec((C, 1), lambda i: (0, 0)),
        ],
        out_specs=[
            pl.BlockSpec((C, _BM), lambda i: (0, i)),
            pl.BlockSpec((C, _BM), lambda i: (0, i)),
            pl.BlockSpec((1, _BM), lambda i: (0, i)),
        ],
        out_shape=[
            jax.ShapeDtypeStruct((C, N), jnp.float32),
            jax.ShapeDtypeStruct((C, N), jnp.float32),
            jax.ShapeDtypeStruct((1, N), jnp.int32),
        ],
        compiler_params=pltpu.CompilerParams(
            dimension_semantics=("parallel",),
        ),
    )(h2, W, bt)

    depth_values = depthT.reshape(B, S)
    last_loss = jnp.zeros((), dtype=jnp.float32)
    return (depth_values, probsT.T.reshape(B, S, C),
            logitsT.T.reshape(B, S, C), last_loss)
